# baseline (speedup 1.0000x reference)
import jax
import jax.numpy as jnp
import numpy as np
from jax.experimental import pallas as pl
from jax.experimental.pallas import tpu as pltpu

_SIGMA = 4.0
_R = int(np.ceil(3.0 * _SIGMA))
_N = 512
_BLK = 128
_WIN = 160
_NBLK = _N // _BLK

_STARTS = [min(max(ib * _BLK - 16, 0), _N - _WIN) for ib in range(_NBLK)]


def _packed_band():
    x = np.arange(-_R, _R + 1, dtype=np.float64)
    w = np.exp(-0.5 * (x / _SIGMA) ** 2)
    w = w / w.sum()
    b = np.zeros((_N, _N), dtype=np.float64)
    rows = np.arange(_N)
    for t in range(2 * _R + 1):
        cols = np.clip(rows + t - _R, 0, _N - 1)
        np.add.at(b, (rows, cols), w[t])
    bp = np.zeros((_NBLK, _BLK, _WIN), dtype=np.float64)
    for ib, st in enumerate(_STARTS):
        bp[ib] = b[ib * _BLK:(ib + 1) * _BLK, st:st + _WIN]
    return bp


_BP = _packed_band()


def _blur_body(img_ref, bp_hbm, out_hbm, bp_v, s_ref, o_ref,
               sem_b, sem0, sem1, sem2, sem3):
    sems = (sem0, sem1, sem2, sem3)
    cp_b = pltpu.make_async_copy(bp_hbm, bp_v, sem_b)
    cp_b.start()
    img16 = img_ref[0].astype(jnp.bfloat16)
    cp_b.wait()
    for ib, st in enumerate(_STARTS):
        s_ref[ib * _BLK:(ib + 1) * _BLK, :] = jax.lax.dot(
            bp_v[ib], img16[st:st + _WIN, :],
            preferred_element_type=jnp.float32).astype(jnp.bfloat16)
    s = s_ref[...]
    copies = []
    for jb, st in enumerate(_STARTS):
        sl = slice(jb * _BLK, (jb + 1) * _BLK)
        o_ref[:, sl] = jax.lax.dot_general(
            s[:, st:st + _WIN], bp_v[jb], (((1,), (1,)), ((), ())),
            preferred_element_type=jnp.float32)
        cp = pltpu.make_async_copy(o_ref.at[:, sl], out_hbm.at[0, :, sl],
                                   sems[jb])
        cp.start()
        copies.append(cp)
    for cp in copies:
        cp.wait()


@jax.jit
def kernel(img):
    return pl.pallas_call(
        _blur_body,
        in_specs=[
            pl.BlockSpec(memory_space=pltpu.MemorySpace.VMEM),
            pl.BlockSpec(memory_space=pltpu.MemorySpace.HBM),
        ],
        out_specs=pl.BlockSpec(memory_space=pltpu.MemorySpace.HBM),
        scratch_shapes=[
            pltpu.VMEM((_NBLK, _BLK, _WIN), jnp.bfloat16),
            pltpu.VMEM((_N, _N), jnp.bfloat16),
            pltpu.VMEM((_N, _N), jnp.float32),
            pltpu.SemaphoreType.DMA,
            pltpu.SemaphoreType.DMA,
            pltpu.SemaphoreType.DMA,
            pltpu.SemaphoreType.DMA,
            pltpu.SemaphoreType.DMA,
        ],
        out_shape=jax.ShapeDtypeStruct((1, _N, _N), jnp.float32),
    )(img, jnp.asarray(_BP, dtype=jnp.bfloat16))


# speedup vs baseline: 1.3202x; 1.3202x over previous
import jax
import jax.numpy as jnp
import numpy as np
from jax.experimental import pallas as pl
from jax.experimental.pallas import tpu as pltpu

_SIGMA = 4.0
_R = int(np.ceil(3.0 * _SIGMA))
_N = 512
_BLK = 128
_WIN = 160
_NBLK = _N // _BLK

_STARTS = [min(max(ib * _BLK - 16, 0), _N - _WIN) for ib in range(_NBLK)]


def _packed_band():
    x = np.arange(-_R, _R + 1, dtype=np.float64)
    w = np.exp(-0.5 * (x / _SIGMA) ** 2)
    w = w / w.sum()
    b = np.zeros((_N, _N), dtype=np.float64)
    rows = np.arange(_N)
    for t in range(2 * _R + 1):
        cols = np.clip(rows + t - _R, 0, _N - 1)
        np.add.at(b, (rows, cols), w[t])
    bp = np.zeros((_NBLK, _BLK, _WIN), dtype=np.float64)
    for ib, st in enumerate(_STARTS):
        bp[ib] = b[ib * _BLK:(ib + 1) * _BLK, st:st + _WIN]
    return bp


_BP = _packed_band()


def _blur_body(img_ref, bp_ref, out_hbm, s_ref, o_ref,
               sem0, sem1, sem2, sem3):
    sems = (sem0, sem1, sem2, sem3)
    img16 = img_ref[0].astype(jnp.bfloat16)
    for ib, st in enumerate(_STARTS):
        s_ref[ib * _BLK:(ib + 1) * _BLK, :] = jax.lax.dot(
            bp_ref[ib], img16[st:st + _WIN, :],
            preferred_element_type=jnp.float32).astype(jnp.bfloat16)
    s = s_ref[...]
    copies = []
    for jb, st in enumerate(_STARTS):
        sl = slice(jb * _BLK, (jb + 1) * _BLK)
        o_ref[:, sl] = jax.lax.dot_general(
            s[:, st:st + _WIN], bp_ref[jb], (((1,), (1,)), ((), ())),
            preferred_element_type=jnp.float32)
        cp = pltpu.make_async_copy(o_ref.at[:, sl], out_hbm.at[0, :, sl],
                                   sems[jb])
        cp.start()
        copies.append(cp)
    for cp in copies:
        cp.wait()


@jax.jit
def kernel(img):
    return pl.pallas_call(
        _blur_body,
        out_specs=pl.BlockSpec(memory_space=pltpu.MemorySpace.HBM),
        scratch_shapes=[
            pltpu.VMEM((_N, _N), jnp.bfloat16),
            pltpu.VMEM((_N, _N), jnp.float32),
            pltpu.SemaphoreType.DMA,
            pltpu.SemaphoreType.DMA,
            pltpu.SemaphoreType.DMA,
            pltpu.SemaphoreType.DMA,
        ],
        out_shape=jax.ShapeDtypeStruct((1, _N, _N), jnp.float32),
    )(img, jnp.asarray(_BP, dtype=jnp.bfloat16))
